# Initial kernel scaffold; baseline (speedup 1.0000x reference)
import jax
import jax.numpy as jnp
from jax.experimental import pallas as pl


def kernel(features, adj, W1, b1, W2, b2, Wy, by):
    raise NotImplementedError("write your pallas kernel here")


# speedup vs baseline: 1.0575x; 1.0575x over previous
import jax
import jax.numpy as jnp
from jax.experimental import pallas as pl


def _prep_kernel(f_ref, w1_ref, b1_ref, t1_ref):
    x = f_ref[...]
    n = jnp.sqrt(jnp.sum(x * x, axis=-1, keepdims=True))
    x = x / jnp.maximum(n, 1e-12)
    t1 = jnp.dot(x, w1_ref[...], preferred_element_type=jnp.float32,
                 precision=jax.lax.Precision.HIGHEST) + b1_ref[...]
    t1_ref[...] = t1.astype(jnp.bfloat16)


def _layer1_kernel(adj_ref, t1_ref, w2_ref, b2_ref, t2_ref):
    a = adj_ref[...].astype(jnp.bfloat16)
    acc = jnp.dot(a, t1_ref[...], preferred_element_type=jnp.float32)
    h1 = jnp.maximum(acc, 0.0)
    t2 = jnp.dot(h1, w2_ref[...], preferred_element_type=jnp.float32,
                 precision=jax.lax.Precision.HIGHEST) + b2_ref[...]
    t2_ref[...] = t2.astype(jnp.bfloat16)


def _layer2_kernel(adj_ref, t2_ref, wy_ref, by_ref, h_ref, y_ref):
    a = adj_ref[...].astype(jnp.bfloat16)
    h2 = jnp.dot(a, t2_ref[...], preferred_element_type=jnp.float32)
    n = jnp.sqrt(jnp.sum(h2 * h2, axis=-1, keepdims=True))
    h = h2 / jnp.maximum(n, 1e-12)
    h_ref[...] = h
    logits = jnp.dot(h, wy_ref[...], preferred_element_type=jnp.float32,
                     precision=jax.lax.Precision.HIGHEST) + by_ref[...]
    m = jnp.max(logits, axis=-1, keepdims=True)
    e = jnp.exp(logits - m)
    y_ref[...] = e / jnp.sum(e, axis=-1, keepdims=True)


def kernel(features, adj, W1, b1, W2, b2, Wy, by):
    N, D = features.shape
    H = W1.shape[1]
    O = W2.shape[1]
    C = Wy.shape[1]
    b1r = b1.reshape(1, H)
    b2r = b2.reshape(1, O)
    byr = by.reshape(1, C)

    t1 = pl.pallas_call(
        _prep_kernel,
        out_shape=jax.ShapeDtypeStruct((N, H), jnp.bfloat16),
    )(features, W1, b1r)

    BM = 400
    grid = (N // BM,)

    t2 = pl.pallas_call(
        _layer1_kernel,
        grid=grid,
        in_specs=[
            pl.BlockSpec((BM, N), lambda i: (i, 0)),
            pl.BlockSpec((N, H), lambda i: (0, 0)),
            pl.BlockSpec((H, O), lambda i: (0, 0)),
            pl.BlockSpec((1, O), lambda i: (0, 0)),
        ],
        out_specs=pl.BlockSpec((BM, O), lambda i: (i, 0)),
        out_shape=jax.ShapeDtypeStruct((N, O), jnp.bfloat16),
    )(adj, t1, W2, b2r)

    h, y = pl.pallas_call(
        _layer2_kernel,
        grid=grid,
        in_specs=[
            pl.BlockSpec((BM, N), lambda i: (i, 0)),
            pl.BlockSpec((N, O), lambda i: (0, 0)),
            pl.BlockSpec((O, C), lambda i: (0, 0)),
            pl.BlockSpec((1, C), lambda i: (0, 0)),
        ],
        out_specs=[
            pl.BlockSpec((BM, H), lambda i: (i, 0)),
            pl.BlockSpec((BM, C), lambda i: (i, 0)),
        ],
        out_shape=[
            jax.ShapeDtypeStruct((N, H), jnp.float32),
            jax.ShapeDtypeStruct((N, C), jnp.float32),
        ],
    )(adj, t2, Wy, byr)

    return (h, y)
